# Initial kernel scaffold; baseline (speedup 1.0000x reference)
import jax
import jax.numpy as jnp
from jax.experimental import pallas as pl


def kernel(input_coord):
    raise NotImplementedError("write your pallas kernel here")


# speedup vs baseline: 5.0864x; 5.0864x over previous
import jax
import jax.numpy as jnp
import numpy as np
from jax.experimental import pallas as pl
from jax.experimental.pallas import tpu as pltpu

N_POINTS = 8192
D_COORD = 128
K = 32
N_OUT = 64
R_BLOCK = 256


def _knn_kernel(sig_ref, x_blk_ref, x_all_ref, vals_ref, idx_ref,
                dist_s, kd_s, ki_s):
    xb = x_blk_ref[...]
    xa = x_all_ref[...]
    sqb = jnp.sum(xb * xb, axis=1)
    sqa = jnp.sum(xa * xa, axis=1)
    prod = jax.lax.dot_general(
        xb, xa, (((1,), (1,)), ((), ())),
        preferred_element_type=jnp.float32)
    d2 = sqb[:, None] + sqa[None, :] - 2.0 * prod
    dist_s[...] = jnp.sqrt(jnp.maximum(d2, 1e-12))

    def body(t, _):
        d = dist_s[...]
        lane = jax.lax.broadcasted_iota(jnp.int32, d.shape, 1)
        m = jnp.min(d, axis=1)
        is_m = d == m[:, None]
        idx = jnp.min(jnp.where(is_m, lane, N_POINTS), axis=1)
        kd_s[pl.ds(t, 1), :] = m[None, :]
        ki_s[pl.ds(t, 1), :] = idx[None, :]
        dist_s[...] = jnp.where(lane == idx[:, None], jnp.inf, d)
        return 0

    jax.lax.fori_loop(0, K, body, 0, unroll=False)

    idx_ref[...] = ki_s[...]
    kd = kd_s[...].T
    sig = sig_ref[0, :]
    dr = jnp.exp(-(kd[:, :, None] ** 2) / (sig[None, None, :] ** 2 * 2.0))
    vals = jnp.where(dr > 0.1, dr, 0.0)
    vals_ref[...] = vals.reshape(R_BLOCK * K, N_OUT)


def _build_pallas():
    n_blocks = N_POINTS // R_BLOCK
    return pl.pallas_call(
        _knn_kernel,
        grid=(n_blocks,),
        in_specs=[
            pl.BlockSpec((1, N_OUT), lambda i: (0, 0)),
            pl.BlockSpec((R_BLOCK, D_COORD), lambda i: (i, 0)),
            pl.BlockSpec((N_POINTS, D_COORD), lambda i: (0, 0)),
        ],
        out_specs=[
            pl.BlockSpec((R_BLOCK * K, N_OUT), lambda i: (i, 0)),
            pl.BlockSpec((K, R_BLOCK), lambda i: (0, i)),
        ],
        out_shape=[
            jax.ShapeDtypeStruct((N_POINTS * K, N_OUT), jnp.float32),
            jax.ShapeDtypeStruct((K, N_POINTS), jnp.int32),
        ],
        scratch_shapes=[
            pltpu.VMEM((R_BLOCK, N_POINTS), jnp.float32),
            pltpu.VMEM((K, R_BLOCK), jnp.float32),
            pltpu.VMEM((K, R_BLOCK), jnp.int32),
        ],
        compiler_params=pltpu.CompilerParams(
            dimension_semantics=("arbitrary",),
        ),
    )


def kernel(input_coord):
    sig_range = jnp.linspace(0.5, 5.0, N_OUT, dtype=jnp.float32)[None, :]
    vals, ki = _build_pallas()(sig_range, input_coord, input_coord)
    col = ki.T.reshape(-1).astype(jnp.int64)
    row = jnp.repeat(jnp.arange(N_POINTS, dtype=jnp.int64), K)
    batch = jnp.zeros_like(col)
    indices = jnp.stack([batch, row, col], axis=0)
    return indices, vals


# speedup vs baseline: 8.0999x; 1.5925x over previous
import jax
import jax.numpy as jnp
import numpy as np
from jax.experimental import pallas as pl
from jax.experimental.pallas import tpu as pltpu

N_POINTS = 8192
D_COORD = 128
K = 32
N_OUT = 64
R_BLOCK = 256
G = 256
C = N_POINTS // G
T = 4


def _knn_kernel(sig_ref, x_blk_ref, x_all_ref, vals_ref, idx_ref,
                dist_s, cv_s, ci_s, kd_s, ki_s):
    xb = x_blk_ref[...]
    xa = x_all_ref[...]
    sqb = jnp.sum(xb * xb, axis=1)
    sqa = jnp.sum(xa * xa, axis=1)
    prod = jax.lax.dot_general(
        xb, xa, (((1,), (1,)), ((), ())),
        preferred_element_type=jnp.float32)
    d2 = sqb[:, None] + sqa[None, :] - 2.0 * prod
    dist_s[...] = jnp.sqrt(jnp.maximum(d2, 1e-12))

    INF = jnp.float32(jnp.inf)

    g_iota = jax.lax.broadcasted_iota(jnp.int32, (R_BLOCK, G), 1)
    acc_v = [jnp.full((R_BLOCK, G), INF, jnp.float32) for _ in range(T)]
    acc_i = [jnp.full((R_BLOCK, G), N_POINTS, jnp.int32) for _ in range(T)]
    for c in range(C):
        t_v = dist_s[:, c * G:(c + 1) * G]
        t_i = g_iota + (c * G)
        for lvl in range(T):
            swap = t_v < acc_v[lvl]
            nv = jnp.where(swap, t_v, acc_v[lvl])
            t_v = jnp.where(swap, acc_v[lvl], t_v)
            ni = jnp.where(swap, t_i, acc_i[lvl])
            t_i = jnp.where(swap, acc_i[lvl], t_i)
            acc_v[lvl] = nv
            acc_i[lvl] = ni
    for lvl in range(T):
        cv_s[:, lvl * G:(lvl + 1) * G] = acc_v[lvl]
        ci_s[:, lvl * G:(lvl + 1) * G] = acc_i[lvl]
    v_last = acc_v[T - 1]

    def body(t, _):
        cv = cv_s[...]
        ci = ci_s[...]
        m = jnp.min(cv, axis=1)
        eq = cv == m[:, None]
        sel = jnp.min(jnp.where(eq, ci, N_POINTS), axis=1)
        kd_s[pl.ds(t, 1), :] = m[None, :]
        ki_s[pl.ds(t, 1), :] = sel[None, :]
        cv_s[...] = jnp.where(eq & (ci == sel[:, None]), INF, cv)
        return 0

    jax.lax.fori_loop(0, K, body, 0, unroll=False)

    tau = kd_s[K - 1, :]
    bad = jnp.any(v_last <= tau[:, None])

    @pl.when(bad)
    def _fallback():
        def fb_body(t, _):
            d = dist_s[...]
            lane = jax.lax.broadcasted_iota(jnp.int32, d.shape, 1)
            m = jnp.min(d, axis=1)
            is_m = d == m[:, None]
            idx = jnp.min(jnp.where(is_m, lane, N_POINTS), axis=1)
            kd_s[pl.ds(t, 1), :] = m[None, :]
            ki_s[pl.ds(t, 1), :] = idx[None, :]
            dist_s[...] = jnp.where(lane == idx[:, None], INF, d)
            return 0
        jax.lax.fori_loop(0, K, fb_body, 0, unroll=False)

    idx_ref[...] = ki_s[...]
    kd = kd_s[...].T
    sig = sig_ref[0, :]
    dr = jnp.exp(-(kd[:, :, None] ** 2) / (sig[None, None, :] ** 2 * 2.0))
    vals = jnp.where(dr > 0.1, dr, 0.0)
    vals_ref[...] = vals.reshape(R_BLOCK * K, N_OUT)


def _build_pallas():
    n_blocks = N_POINTS // R_BLOCK
    return pl.pallas_call(
        _knn_kernel,
        grid=(n_blocks,),
        in_specs=[
            pl.BlockSpec((1, N_OUT), lambda i: (0, 0)),
            pl.BlockSpec((R_BLOCK, D_COORD), lambda i: (i, 0)),
            pl.BlockSpec((N_POINTS, D_COORD), lambda i: (0, 0)),
        ],
        out_specs=[
            pl.BlockSpec((R_BLOCK * K, N_OUT), lambda i: (i, 0)),
            pl.BlockSpec((K, R_BLOCK), lambda i: (0, i)),
        ],
        out_shape=[
            jax.ShapeDtypeStruct((N_POINTS * K, N_OUT), jnp.float32),
            jax.ShapeDtypeStruct((K, N_POINTS), jnp.int32),
        ],
        scratch_shapes=[
            pltpu.VMEM((R_BLOCK, N_POINTS), jnp.float32),
            pltpu.VMEM((R_BLOCK, T * G), jnp.float32),
            pltpu.VMEM((R_BLOCK, T * G), jnp.int32),
            pltpu.VMEM((K, R_BLOCK), jnp.float32),
            pltpu.VMEM((K, R_BLOCK), jnp.int32),
        ],
        compiler_params=pltpu.CompilerParams(
            dimension_semantics=("arbitrary",),
        ),
    )


def kernel(input_coord):
    sig_range = jnp.linspace(0.5, 5.0, N_OUT, dtype=jnp.float32)[None, :]
    vals, ki = _build_pallas()(sig_range, input_coord, input_coord)
    col = ki.T.reshape(-1).astype(jnp.int64)
    row = jnp.repeat(jnp.arange(N_POINTS, dtype=jnp.int64), K)
    batch = jnp.zeros_like(col)
    indices = jnp.stack([batch, row, col], axis=0)
    return indices, vals
